# Initial kernel scaffold; baseline (speedup 1.0000x reference)
import functools
import math

import jax
import jax.numpy as jnp
from jax import lax
from jax.experimental import pallas as pl
from jax.experimental.pallas import tpu as pltpu
from jax.experimental.pallas import tpu_sc as plsc

DEPTH = 5
RATIO = 0.5
NUM_SC = 2
NUM_TILES = 16
NUM_W = NUM_SC * NUM_TILES


def _round_up(a, b):
    return (a + b - 1) // b * b


def _mm_body(a_ref, w_ref, deg_ref, o_ref):
    acc = jnp.dot(a_ref[...], w_ref[...], preferred_element_type=jnp.float32)
    rs = 1.0 / jnp.sqrt(deg_ref[...])
    o_ref[...] = acc * rs


def _mm_prescale(a, w, deg_col, bm=512):
    mpad, k = a.shape
    f = w.shape[1]
    return pl.pallas_call(
        _mm_body,
        grid=(mpad // bm,),
        in_specs=[
            pl.BlockSpec((bm, k), lambda i: (i, 0)),
            pl.BlockSpec((k, f), lambda i: (0, 0)),
            pl.BlockSpec((bm, 1), lambda i: (i, 0)),
        ],
        out_specs=pl.BlockSpec((bm, f), lambda i: (i, 0)),
        out_shape=jax.ShapeDtypeStruct((mpad, f), jnp.float32),
    )(a, w, deg_col)


def _combine_body(acc_ref, hp_ref, deg_ref, b_ref, o_ref, *, relu):
    s = acc_ref[0] + acc_ref[1] + 2.0 * hp_ref[...]
    rs = 1.0 / jnp.sqrt(deg_ref[...])
    r = s * rs + b_ref[...]
    if relu:
        r = jnp.maximum(r, 0.0)
    o_ref[...] = r


def _combine(acc, hp, deg_col, b, n, relu, bm=512):
    f = hp.shape[1]
    return pl.pallas_call(
        functools.partial(_combine_body, relu=relu),
        grid=(_round_up(n, bm) // bm,),
        in_specs=[
            pl.BlockSpec((2, bm, f), lambda i: (0, i, 0)),
            pl.BlockSpec((bm, f), lambda i: (i, 0)),
            pl.BlockSpec((bm, 1), lambda i: (i, 0)),
            pl.BlockSpec((1, f), lambda i: (0, 0)),
        ],
        out_specs=pl.BlockSpec((bm, f), lambda i: (i, 0)),
        out_shape=jax.ShapeDtypeStruct((n, f), jnp.float32),
    )(acc, hp, deg_col, b.reshape(1, f))


def _agg_body(hp, srcp, dstp, nbv, out, idx_s, idx_d, rows, nb_v, sem,
              acc_sp, *, npad, f, be):
    cid = lax.axis_index("c")
    sid = lax.axis_index("s")
    wid = cid + NUM_SC * sid

    def zrow(r, _):
        for c in range(f // 16):
            rows[r, pl.ds(c * 16, 16)] = jnp.zeros((16,), jnp.float32)
        return 0

    lax.fori_loop(0, be, zrow, 0)
    rpt = npad // NUM_TILES

    def zcp(j, _):
        pltpu.sync_copy(rows, acc_sp.at[pl.ds(sid * rpt + j * be, be)])
        return 0

    lax.fori_loop(0, rpt // be, zcp, 0)
    plsc.subcore_barrier()

    pltpu.sync_copy(nbv, nb_v)
    nb = nb_v[0]
    nmine = (nb - wid + NUM_W - 1) // NUM_W

    def body(i, _):
        base = (wid + i * NUM_W) * be
        pltpu.sync_copy(srcp.at[pl.ds(base, be)], idx_s)
        pltpu.sync_copy(dstp.at[pl.ds(base, be)], idx_d)
        pltpu.async_copy(hp.at[idx_s], rows, sem).wait()
        pltpu.sync_copy(rows, acc_sp.at[idx_d], add=True)
        return 0

    lax.fori_loop(0, nmine, body, 0)
    plsc.subcore_barrier()

    def dump(j, _):
        r0 = sid * rpt + j * be
        pltpu.sync_copy(acc_sp.at[pl.ds(r0, be)], rows)
        pltpu.sync_copy(rows, out.at[cid, pl.ds(r0, be)])
        return 0

    lax.fori_loop(0, rpt // be, dump, 0)


def _agg(hp, srcp, dstp, nbv, npad, f, be):
    mesh = plsc.VectorSubcoreMesh(core_axis_name="c", subcore_axis_name="s")
    body = functools.partial(_agg_body, npad=npad, f=f, be=be)
    return pl.kernel(
        body,
        out_type=jax.ShapeDtypeStruct((NUM_SC, npad, f), jnp.float32),
        mesh=mesh,
        scratch_types=[
            pltpu.VMEM((be,), jnp.int32),
            pltpu.VMEM((be,), jnp.int32),
            pltpu.VMEM((be, f), jnp.float32),
            pltpu.VMEM((8,), jnp.int32),
            pltpu.SemaphoreType.DMA,
            pltpu.VMEM_SHARED((npad, f), jnp.float32),
        ],
    )(hp, srcp, dstp, nbv)


def _gcn(x_pad, w, b, deg_col, srcp, dstp, nbv, n, npad, be, relu):
    hp = _mm_prescale(x_pad, w, deg_col)
    acc = _agg(hp, srcp, dstp, nbv, npad, w.shape[1], be)
    return _combine(acc, hp, deg_col, b, n, relu)


def _be_for(f):
    return 64 if f >= 1024 else 128


def _pad_rows(a, npad):
    return jnp.pad(a, ((0, npad - a.shape[0]), (0, 0)))


def _deg_col(deg, n, npad):
    return jnp.pad(deg, (0, npad - n), constant_values=1.0).reshape(npad, 1)


def kernel(x, edge_index, Wd, bd, pw, Wu, bu):
    ns = [10000 if i == 0 else None for i in range(DEPTH + 1)]
    ns[0] = x.shape[0]
    for i in range(1, DEPTH + 1):
        ns[i] = int(math.ceil(RATIO * ns[i - 1]))
    e_cap = edge_index.shape[1]
    ar_e = jnp.arange(e_cap, dtype=jnp.int32)

    src = edge_index[0].astype(jnp.int32)
    dst = edge_index[1].astype(jnp.int32)

    n0 = ns[0]
    f0 = Wd[0].shape[1]
    be0 = _be_for(f0)
    npad0 = _round_up(n0 + 1, NUM_TILES * be0)
    deg0 = jnp.zeros((n0,), jnp.float32).at[dst].add(1.0) + 2.0
    nb0 = jnp.full((8,), _round_up(e_cap, be0) // be0, jnp.int32)
    src0 = jnp.where(ar_e < e_cap, src, npad0 - 1)
    xcur = _gcn(_pad_rows(x, npad0), Wd[0], bd[0], _deg_col(deg0, n0, npad0),
                src0, dst, nb0, n0, npad0, be0, relu=True)

    xs = [xcur]
    lvl = [(src, dst, jnp.int32(e_cap), deg0)]
    perms = []
    csrc, cdst, cnt = src, dst, jnp.int32(e_cap)

    for i in range(1, DEPTH + 1):
        n_prev, n_i = ns[i - 1], ns[i]
        w = pw[i - 1]
        score = jnp.tanh((xcur @ w) / jnp.linalg.norm(w))
        vals, perm = lax.top_k(score, n_i)
        xn = xcur[perm] * vals[:, None]
        mapping = jnp.full((n_prev,), -1, jnp.int32).at[perm].set(
            jnp.arange(n_i, dtype=jnp.int32))
        rs = mapping[csrc]
        rd = mapping[cdst]
        valid = (rs >= 0) & (rd >= 0) & (ar_e < cnt)
        pos = jnp.cumsum(valid.astype(jnp.int32)) - 1
        newcnt = jnp.sum(valid.astype(jnp.int32))
        tgt = jnp.where(valid, pos, e_cap)
        csrc = jnp.zeros((e_cap,), jnp.int32).at[tgt].set(rs, mode="drop")
        cdst = jnp.zeros((e_cap,), jnp.int32).at[tgt].set(rd, mode="drop")
        cnt = newcnt
        deg = jnp.zeros((n_i,), jnp.float32).at[
            jnp.where(ar_e < cnt, cdst, n_i)].add(1.0, mode="drop") + 2.0

        fo = Wd[i].shape[1]
        be = _be_for(fo)
        npad = _round_up(n_i + 1, NUM_TILES * be)
        nb = jnp.full((8,), (cnt + be - 1) // be, jnp.int32)
        srcp = jnp.where(ar_e < cnt, csrc, npad - 1)
        xcur = _gcn(_pad_rows(xn, npad), Wd[i], bd[i], _deg_col(deg, n_i, npad),
                    srcp, cdst, nb, n_i, npad, be, relu=True)
        perms.append(perm)
        if i < DEPTH:
            xs.append(xcur)
            lvl.append((csrc, cdst, cnt, deg))

    for i in range(DEPTH):
        j = DEPTH - 1 - i
        res = xs[j]
        n_j = ns[j]
        up = jnp.zeros((n_j, xcur.shape[1]), jnp.float32).at[perms[j]].set(xcur)
        cat = jnp.concatenate([res, up], axis=-1)

        fo = Wu[i].shape[1]
        be = _be_for(fo)
        npad = _round_up(n_j + 1, NUM_TILES * be)
        csrc_j, cdst_j, cnt_j, deg_j = lvl[j]
        nb = jnp.full((8,), (cnt_j + be - 1) // be, jnp.int32)
        srcp = jnp.where(ar_e < cnt_j, csrc_j, npad - 1)
        xcur = _gcn(_pad_rows(cat, npad), Wu[i], bu[i],
                    _deg_col(deg_j, n_j, npad), srcp, cdst_j, nb,
                    n_j, npad, be, relu=(i < DEPTH - 1))

    return xcur


# speedup vs baseline: 1.1432x; 1.1432x over previous
import functools
import math

import jax
import jax.numpy as jnp
from jax import lax
from jax.experimental import pallas as pl
from jax.experimental.pallas import tpu as pltpu
from jax.experimental.pallas import tpu_sc as plsc

DEPTH = 5
RATIO = 0.5
NUM_SC = 2
NUM_TILES = 16
ROW_ALIGN = 512


def _round_up(a, b):
    return (a + b - 1) // b * b


def _mm_body(a_ref, w_ref, deg_ref, o_ref):
    acc = jnp.dot(a_ref[...], w_ref[...], preferred_element_type=jnp.float32)
    rs = 1.0 / jnp.sqrt(deg_ref[...])
    o_ref[...] = acc * rs


def _mm_prescale(a, w, deg_col, bm=ROW_ALIGN):
    mpad, k = a.shape
    f = w.shape[1]
    return pl.pallas_call(
        _mm_body,
        grid=(mpad // bm,),
        in_specs=[
            pl.BlockSpec((bm, k), lambda i: (i, 0)),
            pl.BlockSpec((k, f), lambda i: (0, 0)),
            pl.BlockSpec((bm, 1), lambda i: (i, 0)),
        ],
        out_specs=pl.BlockSpec((bm, f), lambda i: (i, 0)),
        out_shape=jax.ShapeDtypeStruct((mpad, f), jnp.float32),
    )(a, w, deg_col)


def _combine_body(acc_ref, hp_ref, deg_ref, b_ref, o_ref, *, relu):
    s = acc_ref[...] + 2.0 * hp_ref[...]
    rs = 1.0 / jnp.sqrt(deg_ref[...])
    r = s * rs + b_ref[...]
    if relu:
        r = jnp.maximum(r, 0.0)
    o_ref[...] = r


def _combine(acc, hp, deg_col, b, n, relu, bm=ROW_ALIGN):
    f = hp.shape[1]
    return pl.pallas_call(
        functools.partial(_combine_body, relu=relu),
        grid=(_round_up(n, bm) // bm,),
        in_specs=[
            pl.BlockSpec((bm, f), lambda i: (i, 0)),
            pl.BlockSpec((bm, f), lambda i: (i, 0)),
            pl.BlockSpec((bm, 1), lambda i: (i, 0)),
            pl.BlockSpec((1, f), lambda i: (0, 0)),
        ],
        out_specs=pl.BlockSpec((bm, f), lambda i: (i, 0)),
        out_shape=jax.ShapeDtypeStruct((n, f), jnp.float32),
    )(acc, hp, deg_col, b.reshape(1, f))


def _agg_body(hp, srcp, dstp, nbv, out, idx_s, idx_d, rows, nb_v, sem,
              acc_sp, *, npad, f, be, ecap):
    cid = lax.axis_index("c")
    sid = lax.axis_index("s")
    half = npad // NUM_SC

    def zrow(r, _):
        for c in range(f // 16):
            rows[r, pl.ds(c * 16, 16)] = jnp.zeros((16,), jnp.float32)
        return 0

    lax.fori_loop(0, 16, zrow, 0)
    rpt = half // NUM_TILES

    def zcp(j, _):
        pltpu.sync_copy(rows.at[pl.ds(0, 16)],
                        acc_sp.at[pl.ds(sid * rpt + j * 16, 16)])
        return 0

    lax.fori_loop(0, rpt // 16, zcp, 0)
    plsc.subcore_barrier()

    pltpu.sync_copy(nbv, nb_v)
    nbs = nb_v[...]
    nb = jnp.where(cid == 0, nbs[0], nbs[1])
    nmine = (nb - sid + NUM_TILES - 1) // NUM_TILES
    rebase = cid * half

    def body(i, _):
        base = cid * ecap + (sid + i * NUM_TILES) * be
        pltpu.sync_copy(srcp.at[pl.ds(base, be)], idx_s)
        pltpu.sync_copy(dstp.at[pl.ds(base, be)], idx_d)
        for v in range(be // 16):
            sl = pl.ds(v * 16, 16)
            idx_d[sl] = idx_d[sl] - rebase
        pltpu.async_copy(hp.at[idx_s], rows, sem).wait()
        pltpu.sync_copy(rows, acc_sp.at[idx_d], add=True)
        return 0

    lax.fori_loop(0, nmine, body, 0)
    plsc.subcore_barrier()

    def dump(j, _):
        r0 = sid * rpt + j * 16
        pltpu.sync_copy(acc_sp.at[pl.ds(r0, 16)], rows.at[pl.ds(0, 16)])
        pltpu.sync_copy(rows.at[pl.ds(0, 16)], out.at[cid, pl.ds(r0, 16)])
        return 0

    lax.fori_loop(0, rpt // 16, dump, 0)


def _agg(hp, srcp, dstp, nbv, npad, f, be):
    mesh = plsc.VectorSubcoreMesh(core_axis_name="c", subcore_axis_name="s",
                                  num_cores=NUM_SC, num_subcores=NUM_TILES)
    ecap = srcp.shape[0] // NUM_SC
    body = functools.partial(_agg_body, npad=npad, f=f, be=be, ecap=ecap)
    out = pl.kernel(
        body,
        out_type=jax.ShapeDtypeStruct((NUM_SC, npad // NUM_SC, f),
                                      jnp.float32),
        mesh=mesh,
        compiler_params=pltpu.CompilerParams(use_tc_tiling_on_sc=False),
        scratch_types=[
            pltpu.VMEM((be,), jnp.int32),
            pltpu.VMEM((be,), jnp.int32),
            pltpu.VMEM((be, f), jnp.float32),
            pltpu.VMEM((16,), jnp.int32),
            pltpu.SemaphoreType.DMA,
            pltpu.VMEM_SHARED((npad // NUM_SC, f), jnp.float32),
        ],
    )(hp, srcp, dstp, nbv)
    return out.reshape(npad, f)


def _gcn(x_pad, w, b, deg_col, srcp, dstp, nbv, n, npad, be, relu):
    hp = _mm_prescale(x_pad, w, deg_col)
    acc = _agg(hp, srcp, dstp, nbv, npad, w.shape[1], be)
    return _combine(acc, hp, deg_col, b, n, relu)


def _be_for(f):
    return 64 if f >= 1024 else 128


def _pad_rows(a, npad):
    return jnp.pad(a, ((0, npad - a.shape[0]), (0, 0)))


def _deg_col(deg, n, npad):
    return jnp.pad(deg, (0, npad - n), constant_values=1.0).reshape(npad, 1)


def _bucket(src, dst, valid, e_cap, npad):
    half = npad // NUM_SC
    low = dst < half
    m0 = valid & low
    m1 = valid & ~low
    p0 = jnp.cumsum(m0.astype(jnp.int32)) - 1
    p1 = jnp.cumsum(m1.astype(jnp.int32)) - 1
    cnt0 = jnp.sum(m0.astype(jnp.int32))
    cnt1 = jnp.sum(m1.astype(jnp.int32))
    drop = 2 * e_cap
    tgt = jnp.where(m0, p0, jnp.where(m1, e_cap + p1, drop))
    bsrc = jnp.full((2 * e_cap,), npad - 1, jnp.int32).at[tgt].set(
        src, mode="drop")
    ar2 = jnp.arange(2 * e_cap, dtype=jnp.int32)
    fill = jnp.where(ar2 < e_cap, 0, half)
    bdst = fill.at[tgt].set(dst, mode="drop")
    return bsrc, bdst, cnt0, cnt1


def _nbv(cnt0, cnt1, be):
    return jnp.stack([(cnt0 + be - 1) // be, (cnt1 + be - 1) // be] +
                     [jnp.int32(0)] * 14).astype(jnp.int32)


def kernel(x, edge_index, Wd, bd, pw, Wu, bu):
    ns = [x.shape[0]]
    for _ in range(DEPTH):
        ns.append(int(math.ceil(RATIO * ns[-1])))
    e_cap = edge_index.shape[1]
    ar2 = jnp.arange(2 * e_cap, dtype=jnp.int32)
    in_bucket = lambda c0, c1: jnp.where(ar2 < e_cap, ar2 < c0,
                                         (ar2 - e_cap) < c1)

    src = edge_index[0].astype(jnp.int32)
    dst = edge_index[1].astype(jnp.int32)

    n0 = ns[0]
    f0 = Wd[0].shape[1]
    npad0 = _round_up(n0 + 1, ROW_ALIGN)
    deg0 = jnp.zeros((n0,), jnp.float32).at[dst].add(1.0) + 2.0
    bsrc, bdst, cnt0, cnt1 = _bucket(
        src, dst, jnp.ones((e_cap,), bool), e_cap, npad0)
    be0 = _be_for(f0)
    xcur = _gcn(_pad_rows(x, npad0), Wd[0], bd[0], _deg_col(deg0, n0, npad0),
                bsrc, bdst, _nbv(cnt0, cnt1, be0), n0, npad0, be0, relu=True)

    xs = [xcur]
    lvl = [(bsrc, bdst, cnt0, cnt1, deg0, npad0)]
    perms = []

    for i in range(1, DEPTH + 1):
        n_prev, n_i = ns[i - 1], ns[i]
        bsrc_p, bdst_p, c0_p, c1_p, _, _ = lvl[i - 1]
        w = pw[i - 1]
        score = jnp.tanh((xcur @ w) / jnp.linalg.norm(w))
        vals, perm = lax.top_k(score, n_i)
        xn = xcur[perm] * vals[:, None]
        mapping = jnp.full((n_prev,), -1, jnp.int32).at[perm].set(
            jnp.arange(n_i, dtype=jnp.int32))
        rs = mapping[jnp.clip(bsrc_p, 0, n_prev - 1)]
        rd = mapping[jnp.clip(bdst_p, 0, n_prev - 1)]
        valid = (rs >= 0) & (rd >= 0) & in_bucket(c0_p, c1_p)

        fo = Wd[i].shape[1]
        be = _be_for(fo)
        npad = _round_up(n_i + 1, ROW_ALIGN)
        bsrc, bdst, cnt0, cnt1 = _bucket(rs, rd, valid, e_cap, npad)
        deg = jnp.zeros((n_i,), jnp.float32).at[
            jnp.where(valid, rd, n_i)].add(1.0, mode="drop") + 2.0

        xcur = _gcn(_pad_rows(xn, npad), Wd[i], bd[i],
                    _deg_col(deg, n_i, npad), bsrc, bdst,
                    _nbv(cnt0, cnt1, be), n_i, npad, be, relu=True)
        perms.append(perm)
        if i < DEPTH:
            xs.append(xcur)
            lvl.append((bsrc, bdst, cnt0, cnt1, deg, npad))

    for i in range(DEPTH):
        j = DEPTH - 1 - i
        res = xs[j]
        n_j = ns[j]
        up = jnp.zeros((n_j, xcur.shape[1]), jnp.float32).at[perms[j]].set(xcur)
        cat = jnp.concatenate([res, up], axis=-1)

        fo = Wu[i].shape[1]
        be = _be_for(fo)
        bsrc_j, bdst_j, c0_j, c1_j, deg_j, npad_j = lvl[j]
        xcur = _gcn(_pad_rows(cat, npad_j), Wu[i], bu[i],
                    _deg_col(deg_j, n_j, npad_j), bsrc_j, bdst_j,
                    _nbv(c0_j, c1_j, be), n_j, npad_j, be,
                    relu=(i < DEPTH - 1))

    return xcur
